# Initial kernel scaffold; baseline (speedup 1.0000x reference)
import jax
import jax.numpy as jnp
from jax.experimental import pallas as pl


def kernel(flat, cu_seqlens, centers, log_sharpness):
    raise NotImplementedError("write your pallas kernel here")


# speedup vs baseline: 4.7632x; 4.7632x over previous
import functools

import jax
import jax.numpy as jnp
from jax import lax
from jax.experimental import pallas as pl
from jax.experimental.pallas import tpu as pltpu
from jax.experimental.pallas import tpu_sc as plsc

T = 32768
B = 16
E = 128
D = 2
L = 16
NC = 2
NS = 16
NW = NC * NS
CHUNK = T // NW
EV = E // L


def _sc_body(flat_hbm, coef_hbm, bounds_hbm, out_hbm,
             pts_v, coef_v, bounds_v, acc_v, idx_v, shared):
    c = lax.axis_index("c")
    s = lax.axis_index("s")
    wid = c * NS + s
    base = wid * CHUNK

    pltpu.sync_copy(flat_hbm.at[pl.ds(base * D, CHUNK * D)], pts_v)
    pltpu.sync_copy(coef_hbm, coef_v)
    pltpu.sync_copy(bounds_hbm.at[wid], bounds_v)

    zero = jnp.zeros((L,), jnp.float32)
    for b in range(B):
        for j in range(EV):
            acc_v[b, pl.ds(j * L, L)] = zero
    idx_v[...] = lax.iota(jnp.int32, L)

    @pl.when(s == 0)
    def _():
        pltpu.sync_copy(acc_v, shared)

    plsc.subcore_barrier()

    c0 = [coef_v[0, pl.ds(j * L, L)] for j in range(EV)]
    c1 = [coef_v[1, pl.ds(j * L, L)] for j in range(EV)]
    ns0 = [coef_v[2, pl.ds(j * L, L)] for j in range(EV)]
    ns1 = [coef_v[3, pl.ds(j * L, L)] for j in range(EV)]

    lov = bounds_v[0, :]
    hiv = bounds_v[1, :]
    for b in range(B):
        lo = lov[b]
        hi = hiv[b]

        def body(t, accs):
            vv = pts_v[pl.ds(t * D, L)]
            px = vv[0]
            py = vv[1]
            pxv = jnp.full((L,), px, jnp.float32)
            pyv = jnp.full((L,), py, jnp.float32)
            new = []
            for j in range(EV):
                d0 = pxv - c0[j]
                d1 = pyv - c1[j]
                u = d0 * d0 * ns0[j] + d1 * d1 * ns1[j]
                new.append(accs[j] + jnp.exp(u))
            return tuple(new)

        accs = lax.fori_loop(
            lo, hi, body,
            tuple(jnp.zeros((L,), jnp.float32) for _ in range(EV)))
        for j in range(EV):
            plsc.addupdate(acc_v.at[b, pl.ds(j * L, L)], accs[j])

    pltpu.sync_copy(acc_v, shared.at[idx_v], add=True)
    plsc.subcore_barrier()

    @pl.when(s == 0)
    def _():
        pltpu.sync_copy(shared, out_hbm.at[c])


_sc_call = functools.partial(
    pl.kernel,
    out_type=jax.ShapeDtypeStruct((NC, B, E), jnp.float32),
    mesh=plsc.VectorSubcoreMesh(
        core_axis_name="c", subcore_axis_name="s",
        num_cores=NC, num_subcores=NS),
    scratch_types=[
        pltpu.VMEM((CHUNK * D,), jnp.float32),
        pltpu.VMEM((4, E), jnp.float32),
        pltpu.VMEM((2, B), jnp.int32),
        pltpu.VMEM((B, E), jnp.float32),
        pltpu.VMEM((L,), jnp.int32),
        pltpu.VMEM_SHARED((B, E), jnp.float32),
    ],
)(_sc_body)


def kernel(flat, cu_seqlens, centers, log_sharpness):
    sharp = jax.nn.softplus(log_sharpness) + 1e-06
    coef = jnp.stack(
        [centers[:, 0], centers[:, 1], -sharp[:, 0], -sharp[:, 1]])
    cu = cu_seqlens.astype(jnp.int32)
    wbase = jnp.arange(NW, dtype=jnp.int32)[:, None] * CHUNK
    lo = jnp.clip(cu[None, :-1] - wbase, 0, CHUNK)
    hi = jnp.clip(cu[None, 1:] - wbase, 0, CHUNK)
    bounds = jnp.stack([lo, hi], axis=1)
    partial = _sc_call(flat.reshape(T * D), coef, bounds)
    return partial[0] + partial[1]


# speedup vs baseline: 5.4291x; 1.1398x over previous
import functools

import jax
import jax.numpy as jnp
from jax import lax
from jax.experimental import pallas as pl
from jax.experimental.pallas import tpu as pltpu
from jax.experimental.pallas import tpu_sc as plsc

T = 32768
B = 16
E = 128
D = 2
L = 16
NC = 2
NS = 16
NW = NC * NS
CHUNK = T // NW
EV = E // L


def _sc_body(px_hbm, py_hbm, coef_hbm, bounds_hbm, out_hbm,
             px_v, py_v, coef_v, bounds_v, acc_v, idx_v, shared):
    c = lax.axis_index("c")
    s = lax.axis_index("s")
    wid = c * NS + s
    base = wid * CHUNK

    pltpu.sync_copy(px_hbm.at[pl.ds(base, CHUNK)], px_v.at[pl.ds(0, CHUNK)])
    pltpu.sync_copy(py_hbm.at[pl.ds(base, CHUNK)], py_v.at[pl.ds(0, CHUNK)])
    pltpu.sync_copy(coef_hbm, coef_v)
    pltpu.sync_copy(bounds_hbm.at[wid], bounds_v)

    zero = jnp.zeros((L,), jnp.float32)
    for b in range(B):
        for j in range(EV):
            acc_v[b, pl.ds(j * L, L)] = zero
    idx_v[...] = lax.iota(jnp.int32, L)

    @pl.when(s == 0)
    def _():
        pltpu.sync_copy(acc_v, shared)

    plsc.subcore_barrier()

    c0 = [coef_v[0, pl.ds(j * L, L)] for j in range(EV)]
    c1 = [coef_v[1, pl.ds(j * L, L)] for j in range(EV)]
    ns0 = [coef_v[2, pl.ds(j * L, L)] for j in range(EV)]
    ns1 = [coef_v[3, pl.ds(j * L, L)] for j in range(EV)]

    lov = bounds_v[0, :]
    hiv = bounds_v[1, :]
    for b in range(B):
        lo = lov[b]
        hi = hiv[b]

        def body(t, accs):
            px = px_v[pl.ds(t, L)][0]
            py = py_v[pl.ds(t, L)][0]
            pxv = jnp.full((L,), px, jnp.float32)
            pyv = jnp.full((L,), py, jnp.float32)
            new = []
            for j in range(EV):
                d0 = pxv - c0[j]
                d1 = pyv - c1[j]
                u = d0 * d0 * ns0[j] + d1 * d1 * ns1[j]
                new.append(accs[j] + jnp.exp(u))
            return tuple(new)

        accs = lax.fori_loop(
            lo, hi, body,
            tuple(jnp.zeros((L,), jnp.float32) for _ in range(EV)))
        for j in range(EV):
            plsc.addupdate(acc_v.at[b, pl.ds(j * L, L)], accs[j])

    pltpu.sync_copy(acc_v, shared.at[idx_v], add=True)
    plsc.subcore_barrier()

    @pl.when(s == 0)
    def _():
        pltpu.sync_copy(shared, out_hbm.at[c])


_sc_call = functools.partial(
    pl.kernel,
    out_type=jax.ShapeDtypeStruct((NC, B, E), jnp.float32),
    mesh=plsc.VectorSubcoreMesh(
        core_axis_name="c", subcore_axis_name="s",
        num_cores=NC, num_subcores=NS),
    scratch_types=[
        pltpu.VMEM((CHUNK + L,), jnp.float32),
        pltpu.VMEM((CHUNK + L,), jnp.float32),
        pltpu.VMEM((4, E), jnp.float32),
        pltpu.VMEM((2, B), jnp.int32),
        pltpu.VMEM((B, E), jnp.float32),
        pltpu.VMEM((L,), jnp.int32),
        pltpu.VMEM_SHARED((B, E), jnp.float32),
    ],
)(_sc_body)


def kernel(flat, cu_seqlens, centers, log_sharpness):
    sharp = jax.nn.softplus(log_sharpness) + 1e-06
    coef = jnp.stack(
        [centers[:, 0], centers[:, 1], -sharp[:, 0], -sharp[:, 1]])
    cu = cu_seqlens.astype(jnp.int32)
    wbase = jnp.arange(NW, dtype=jnp.int32)[:, None] * CHUNK
    lo = jnp.clip(cu[None, :-1] - wbase, 0, CHUNK)
    hi = jnp.clip(cu[None, 1:] - wbase, 0, CHUNK)
    bounds = jnp.stack([lo, hi], axis=1)
    partial = _sc_call(flat[:, 0], flat[:, 1], coef, bounds)
    return partial[0] + partial[1]


# speedup vs baseline: 5.8628x; 1.0799x over previous
import functools

import jax
import jax.numpy as jnp
from jax import lax
from jax.experimental import pallas as pl
from jax.experimental.pallas import tpu as pltpu
from jax.experimental.pallas import tpu_sc as plsc

T = 32768
B = 16
E = 128
D = 2
L = 16
NC = 2
NS = 16
NW = NC * NS
CHUNK = T // NW
EV = E // L
BE = B * E
SLICE = BE // NS
BND = 48


def _sc_body(px_hbm, py_hbm, coef_hbm, bounds_hbm, out_hbm,
             px_v, py_v, coef_v, bnd_v, acc_v, red_v, shared):
    c = lax.axis_index("c")
    s = lax.axis_index("s")
    wid = c * NS + s
    base = wid * CHUNK

    pltpu.sync_copy(px_hbm.at[pl.ds(base, CHUNK)], px_v)
    pltpu.sync_copy(py_hbm.at[pl.ds(base, CHUNK)], py_v)
    pltpu.sync_copy(coef_hbm, coef_v)
    pltpu.sync_copy(bounds_hbm.at[wid], bnd_v)

    zero = jnp.zeros((L,), jnp.float32)
    for k in range(BE // L):
        acc_v[pl.ds(k * L, L)] = zero

    c0 = [coef_v[0, pl.ds(j * L, L)] for j in range(EV)]
    c1 = [coef_v[1, pl.ds(j * L, L)] for j in range(EV)]
    ns0 = [coef_v[2, pl.ds(j * L, L)] for j in range(EV)]
    ns1 = [coef_v[3, pl.ds(j * L, L)] for j in range(EV)]

    iota = lax.iota(jnp.int32, L)
    big = jnp.float32(1e18)

    def seg_body(b, carry):
        lo = bnd_v[pl.ds(b, L)][0]
        hi = bnd_v[pl.ds(b + B, L)][0]
        g0 = lax.shift_right_logical(lo, 4)
        g1 = lax.shift_right_logical(hi + (L - 1), 4)

        def grp_body(g, accs):
            gbase = g * L
            gx = px_v[pl.ds(gbase, L)]
            gy = py_v[pl.ds(gbase, L)]
            tvec = iota + jnp.full((L,), gbase, jnp.int32)
            mask = (tvec >= jnp.full((L,), lo, jnp.int32)) & (
                tvec < jnp.full((L,), hi, jnp.int32))
            gxe = jnp.where(mask, gx, big)
            gye = jnp.where(mask, gy, big)
            new = accs
            for l in range(L):
                pxv = jnp.full((L,), gxe[l], jnp.float32)
                pyv = jnp.full((L,), gye[l], jnp.float32)
                tmp = []
                for j in range(EV):
                    d0 = pxv - c0[j]
                    d1 = pyv - c1[j]
                    u = d0 * d0 * ns0[j] + d1 * d1 * ns1[j]
                    tmp.append(new[j] + jnp.exp(u))
                new = tuple(tmp)
            return new

        accs = lax.fori_loop(
            g0, g1, grp_body,
            tuple(jnp.zeros((L,), jnp.float32) for _ in range(EV)))
        boff = b * E
        for j in range(EV):
            plsc.addupdate(acc_v.at[pl.ds(boff + j * L, L)], accs[j])
        return carry

    lax.fori_loop(0, B, seg_body, jnp.int32(0))

    pltpu.sync_copy(acc_v, shared.at[s])
    plsc.subcore_barrier()
    soff = s * SLICE
    for j in range(SLICE // L):
        red_v[pl.ds(j * L, L)] = zero
    for r in range(NS):
        row = shared.at[r, pl.ds(soff, SLICE)]
        pltpu.sync_copy(row, red_v.at[pl.ds(SLICE, SLICE)])
        for j in range(SLICE // L):
            plsc.addupdate(red_v.at[pl.ds(j * L, L)],
                           red_v[pl.ds(SLICE + j * L, L)])
    pltpu.sync_copy(red_v.at[pl.ds(0, SLICE)], out_hbm.at[c, pl.ds(soff, SLICE)])


_sc_call = functools.partial(
    pl.kernel,
    out_type=jax.ShapeDtypeStruct((NC, BE), jnp.float32),
    mesh=plsc.VectorSubcoreMesh(
        core_axis_name="c", subcore_axis_name="s",
        num_cores=NC, num_subcores=NS),
    scratch_types=[
        pltpu.VMEM((CHUNK,), jnp.float32),
        pltpu.VMEM((CHUNK,), jnp.float32),
        pltpu.VMEM((4, E), jnp.float32),
        pltpu.VMEM((BND,), jnp.int32),
        pltpu.VMEM((BE,), jnp.float32),
        pltpu.VMEM((2 * SLICE,), jnp.float32),
        pltpu.VMEM_SHARED((NS, BE), jnp.float32),
    ],
)(_sc_body)


def kernel(flat, cu_seqlens, centers, log_sharpness):
    sharp = jax.nn.softplus(log_sharpness) + 1e-06
    coef = jnp.stack(
        [centers[:, 0], centers[:, 1], -sharp[:, 0], -sharp[:, 1]])
    cu = cu_seqlens.astype(jnp.int32)
    wbase = jnp.arange(NW, dtype=jnp.int32)[:, None] * CHUNK
    lo = jnp.clip(cu[None, :-1] - wbase, 0, CHUNK)
    hi = jnp.clip(cu[None, 1:] - wbase, 0, CHUNK)
    pad = jnp.zeros((NW, BND - 2 * B), jnp.int32)
    bounds = jnp.concatenate([lo, hi, pad], axis=1)
    partial = _sc_call(flat[:, 0], flat[:, 1], coef, bounds)
    return (partial[0] + partial[1]).reshape(B, E)


# speedup vs baseline: 6.4010x; 1.0918x over previous
import functools

import jax
import jax.numpy as jnp
from jax import lax
from jax.experimental import pallas as pl
from jax.experimental.pallas import tpu as pltpu
from jax.experimental.pallas import tpu_sc as plsc

T = 32768
B = 16
E = 128
D = 2
L = 16
NC = 2
NS = 16
NW = NC * NS
CHUNK = T // NW
EV = E // L
BE = B * E
SLICE = BE // NS
BND = 48


def _sc_body(px_hbm, py_hbm, coef_hbm, bounds_hbm, out_hbm,
             px_v, py_v, coef_v, bnd_v, acc_v, red_v, shared):
    c = lax.axis_index("c")
    s = lax.axis_index("s")
    wid = c * NS + s
    base = wid * CHUNK

    pltpu.sync_copy(px_hbm.at[pl.ds(base, CHUNK)], px_v.at[pl.ds(0, CHUNK)])
    pltpu.sync_copy(py_hbm.at[pl.ds(base, CHUNK)], py_v.at[pl.ds(0, CHUNK)])
    pltpu.sync_copy(coef_hbm, coef_v)
    pltpu.sync_copy(bounds_hbm.at[wid], bnd_v)

    zero = jnp.zeros((L,), jnp.float32)
    for k in range(BE // L):
        acc_v[pl.ds(k * L, L)] = zero

    c0 = [coef_v[0, pl.ds(j * L, L)] for j in range(EV)]
    c1 = [coef_v[1, pl.ds(j * L, L)] for j in range(EV)]
    ns0 = [coef_v[2, pl.ds(j * L, L)] for j in range(EV)]
    ns1 = [coef_v[3, pl.ds(j * L, L)] for j in range(EV)]

    def seg_body(b, carry):
        lo = bnd_v[pl.ds(b, L)][0]
        hi = bnd_v[pl.ds(b + B, L)][0]

        def body(t, accs):
            pxv = jnp.full((L,), px_v[pl.ds(t, L)][0], jnp.float32)
            pyv = jnp.full((L,), py_v[pl.ds(t, L)][0], jnp.float32)
            new = []
            for j in range(EV):
                d0 = pxv - c0[j]
                d1 = pyv - c1[j]
                u = d0 * d0 * ns0[j] + d1 * d1 * ns1[j]
                new.append(accs[j] + jnp.exp(u))
            return tuple(new)

        accs = lax.fori_loop(
            lo, hi, body,
            tuple(jnp.zeros((L,), jnp.float32) for _ in range(EV)))
        boff = b * E
        for j in range(EV):
            plsc.addupdate(acc_v.at[pl.ds(boff + j * L, L)], accs[j])
        return carry

    lax.fori_loop(0, B, seg_body, jnp.int32(0))

    pltpu.sync_copy(acc_v, shared.at[s])
    plsc.subcore_barrier()
    soff = s * SLICE
    for j in range(SLICE // L):
        red_v[pl.ds(j * L, L)] = zero
    for r in range(NS):
        row = shared.at[r, pl.ds(soff, SLICE)]
        pltpu.sync_copy(row, red_v.at[pl.ds(SLICE, SLICE)])
        for j in range(SLICE // L):
            plsc.addupdate(red_v.at[pl.ds(j * L, L)],
                           red_v[pl.ds(SLICE + j * L, L)])
    pltpu.sync_copy(red_v.at[pl.ds(0, SLICE)], out_hbm.at[c, pl.ds(soff, SLICE)])


_sc_call = functools.partial(
    pl.kernel,
    out_type=jax.ShapeDtypeStruct((NC, BE), jnp.float32),
    mesh=plsc.VectorSubcoreMesh(
        core_axis_name="c", subcore_axis_name="s",
        num_cores=NC, num_subcores=NS),
    scratch_types=[
        pltpu.VMEM((CHUNK + L,), jnp.float32),
        pltpu.VMEM((CHUNK + L,), jnp.float32),
        pltpu.VMEM((4, E), jnp.float32),
        pltpu.VMEM((BND,), jnp.int32),
        pltpu.VMEM((BE,), jnp.float32),
        pltpu.VMEM((2 * SLICE,), jnp.float32),
        pltpu.VMEM_SHARED((NS, BE), jnp.float32),
    ],
)(_sc_body)


def kernel(flat, cu_seqlens, centers, log_sharpness):
    sharp = jax.nn.softplus(log_sharpness) + 1e-06
    coef = jnp.stack(
        [centers[:, 0], centers[:, 1], -sharp[:, 0], -sharp[:, 1]])
    cu = cu_seqlens.astype(jnp.int32)
    wbase = jnp.arange(NW, dtype=jnp.int32)[:, None] * CHUNK
    lo = jnp.clip(cu[None, :-1] - wbase, 0, CHUNK)
    hi = jnp.clip(cu[None, 1:] - wbase, 0, CHUNK)
    pad = jnp.zeros((NW, BND - 2 * B), jnp.int32)
    bounds = jnp.concatenate([lo, hi, pad], axis=1)
    partial = _sc_call(flat[:, 0], flat[:, 1], coef, bounds)
    return (partial[0] + partial[1]).reshape(B, E)


# speedup vs baseline: 6.8556x; 1.0710x over previous
import functools

import jax
import jax.numpy as jnp
from jax import lax
from jax.experimental import pallas as pl
from jax.experimental.pallas import tpu as pltpu
from jax.experimental.pallas import tpu_sc as plsc

T = 32768
B = 16
E = 128
D = 2
L = 16
NC = 2
NS = 16
NW = NC * NS
CHUNK = T // NW
EV = E // L
BE = B * E
SLICE = BE // NS
BND = 48


def _sc_body(px_hbm, py_hbm, coef_hbm, bounds_hbm, out_hbm,
             px_v, py_v, coef_v, bnd_v, acc_v, red_v, red2_v, shared):
    c = lax.axis_index("c")
    s = lax.axis_index("s")
    wid = c * NS + s
    base = wid * CHUNK

    pltpu.sync_copy(px_hbm.at[pl.ds(base, CHUNK)], px_v.at[pl.ds(0, CHUNK)])
    pltpu.sync_copy(py_hbm.at[pl.ds(base, CHUNK)], py_v.at[pl.ds(0, CHUNK)])
    pltpu.sync_copy(coef_hbm, coef_v)
    pltpu.sync_copy(bounds_hbm.at[wid], bnd_v)

    zero = jnp.zeros((L,), jnp.float32)
    for k in range(BE // L):
        acc_v[pl.ds(k * L, L)] = zero

    c0 = [coef_v[0, pl.ds(j * L, L)] for j in range(EV)]
    c1 = [coef_v[1, pl.ds(j * L, L)] for j in range(EV)]
    ns0 = [coef_v[2, pl.ds(j * L, L)] for j in range(EV)]
    ns1 = [coef_v[3, pl.ds(j * L, L)] for j in range(EV)]

    def _resp(pxv, pyv, j):
        d0 = pxv - c0[j]
        d1 = pyv - c1[j]
        return jnp.exp(d0 * d0 * ns0[j] + d1 * d1 * ns1[j])

    def seg_body(b, carry):
        lo = bnd_v[pl.ds(b, L)][0]
        hi = bnd_v[pl.ds(b + B, L)][0]
        boff = b * E

        def _single(t):
            vx = jnp.full((L,), px_v[pl.ds(t, L)][0], jnp.float32)
            vy = jnp.full((L,), py_v[pl.ds(t, L)][0], jnp.float32)
            for j in range(EV):
                plsc.addupdate(acc_v.at[pl.ds(boff + j * L, L)],
                               _resp(vx, vy, j))

        @pl.when(((lo & 1) == 1) & (lo < hi))
        def _():
            _single(lo)

        @pl.when(((hi & 1) == 1) & (hi - 1 >= lo))
        def _():
            _single(hi - 1)

        def body(p, accs):
            vldx = px_v[pl.ds(p * 2, L)]
            vldy = py_v[pl.ds(p * 2, L)]
            x0 = jnp.full((L,), vldx[0], jnp.float32)
            y0 = jnp.full((L,), vldy[0], jnp.float32)
            x1 = jnp.full((L,), vldx[1], jnp.float32)
            y1 = jnp.full((L,), vldy[1], jnp.float32)
            new = []
            for j in range(EV):
                new.append(accs[j] + _resp(x0, y0, j) + _resp(x1, y1, j))
            return tuple(new)

        accs = lax.fori_loop(
            lax.shift_right_logical(lo + 1, 1),
            lax.shift_right_logical(hi, 1),
            body,
            tuple(jnp.zeros((L,), jnp.float32) for _ in range(EV)))
        for j in range(EV):
            plsc.addupdate(acc_v.at[pl.ds(boff + j * L, L)], accs[j])
        return carry

    lax.fori_loop(0, B, seg_body, jnp.int32(0))

    pltpu.sync_copy(acc_v, shared.at[s])
    plsc.subcore_barrier()
    soff = s * SLICE
    pltpu.sync_copy(shared.at[:, pl.ds(soff, SLICE)], red2_v)
    for j in range(SLICE // L):
        tot = red2_v[0, pl.ds(j * L, L)]
        for r in range(1, NS):
            tot = tot + red2_v[r, pl.ds(j * L, L)]
        red_v[pl.ds(j * L, L)] = tot
    pltpu.sync_copy(red_v, out_hbm.at[c, pl.ds(soff, SLICE)])


_sc_call = functools.partial(
    pl.kernel,
    out_type=jax.ShapeDtypeStruct((NC, BE), jnp.float32),
    mesh=plsc.VectorSubcoreMesh(
        core_axis_name="c", subcore_axis_name="s",
        num_cores=NC, num_subcores=NS),
    scratch_types=[
        pltpu.VMEM((CHUNK + L,), jnp.float32),
        pltpu.VMEM((CHUNK + L,), jnp.float32),
        pltpu.VMEM((4, E), jnp.float32),
        pltpu.VMEM((BND,), jnp.int32),
        pltpu.VMEM((BE,), jnp.float32),
        pltpu.VMEM((SLICE,), jnp.float32),
        pltpu.VMEM((NS, SLICE), jnp.float32),
        pltpu.VMEM_SHARED((NS, BE), jnp.float32),
    ],
)(_sc_body)


def kernel(flat, cu_seqlens, centers, log_sharpness):
    sharp = jax.nn.softplus(log_sharpness) + 1e-06
    coef = jnp.stack(
        [centers[:, 0], centers[:, 1], -sharp[:, 0], -sharp[:, 1]])
    cu = cu_seqlens.astype(jnp.int32)
    wbase = jnp.arange(NW, dtype=jnp.int32)[:, None] * CHUNK
    lo = jnp.clip(cu[None, :-1] - wbase, 0, CHUNK)
    hi = jnp.clip(cu[None, 1:] - wbase, 0, CHUNK)
    pad = jnp.zeros((NW, BND - 2 * B), jnp.int32)
    bounds = jnp.concatenate([lo, hi, pad], axis=1)
    partial = _sc_call(flat[:, 0], flat[:, 1], coef, bounds)
    return (partial[0] + partial[1]).reshape(B, E)
